# baseline (speedup 1.0000x reference)
import functools

import jax
import jax.numpy as jnp
from jax import lax
from jax.experimental import pallas as pl
from jax.experimental.pallas import tpu as pltpu
from jax.experimental.pallas import tpu_sc as plsc

_MAXP = 2048
_NH = 16
_TBL = 2 * _MAXP - 1
_TBLP = 4104
_SEQ = 2048
_NC, _NS = 2, 16
_NW = _NC * _NS
_ROWS_PER_W = _SEQ // _NW
_WIN = 2552
_WINP = 2560

_mesh = plsc.VectorSubcoreMesh(core_axis_name="c", subcore_axis_name="s")


@functools.partial(
    pl.kernel,
    mesh=_mesh,
    out_type=jax.ShapeDtypeStruct((_SEQ, 2, 16, 8, 128), jnp.float32),
    scratch_types=[
        pltpu.VMEM((_WIN * _NH,), jnp.float32),
        pltpu.VMEM((2, 8, _WINP), jnp.float32),
        pltpu.SemaphoreType.DMA,
    ],
    compiler_params=pltpu.CompilerParams(
        use_tc_tiling_on_sc=False, needs_layout_passes=False
    ),
)
def _rel_embed(tab_hbm, out_hbm, win_v, revt_v, ssem):
    wid = lax.axis_index("s") * _NC + lax.axis_index("c")
    m = wid % 8
    q = wid // 8
    t0 = m + 512 * q

    pltpu.sync_copy(tab_hbm.at[pl.ds(t0 * _NH, _WIN * _NH)], win_v)

    lane = lax.iota(jnp.int32, 16)
    off0 = lane * _NH

    @plsc.parallel_loop(0, _WINP // 16, step=1, unroll=4)
    def _build(s):
        w0 = s * 16
        base = jnp.maximum(
            jnp.full((16,), (_WIN - 1 - w0) * _NH, jnp.int32) - off0, 0
        )
        for h in range(16):
            vec = plsc.load_gather(win_v, [base + h])
            revt_v[h // 8, h % 8, pl.ds(pl.multiple_of(w0, 16), 16)] = vec

    def _scatter(t, _):
        o = (_WIN - _SEQ) - 8 * t
        i = t0 + 8 * t
        handles = []
        for ab in range(32):
            a = ab // 16
            b = ab % 16
            handles.append(
                pltpu.async_copy(
                    revt_v.at[a, :, pl.ds(pl.multiple_of(o + 128 * b, 8), 128)],
                    out_hbm.at[i, a, b],
                    ssem,
                )
            )
        for h in handles:
            h.wait()
        return 0

    lax.fori_loop(0, _ROWS_PER_W, _scatter, 0)


def kernel(rel_table, seq_len):
    del seq_len
    padded = jnp.concatenate(
        [rel_table, jnp.zeros((_TBLP - _TBL, _NH), rel_table.dtype)], axis=0
    )
    five = _rel_embed(padded.reshape(-1))
    return five.transpose(0, 2, 4, 1, 3).reshape(_SEQ, _SEQ, _NH)


# speedup vs baseline: 1.0174x; 1.0174x over previous
import functools

import jax
import jax.numpy as jnp
from jax import lax
from jax.experimental import pallas as pl
from jax.experimental.pallas import tpu as pltpu
from jax.experimental.pallas import tpu_sc as plsc

_MAXP = 2048
_NH = 16
_TBL = 2 * _MAXP - 1
_TBLP = 4104
_SEQ = 2048
_NC, _NS = 2, 16
_NW = _NC * _NS
_ROWS_PER_W = _SEQ // _NW
_WIN = 2552
_WINP = 2560

_mesh = plsc.VectorSubcoreMesh(core_axis_name="c", subcore_axis_name="s")


@functools.partial(
    pl.kernel,
    mesh=_mesh,
    out_type=jax.ShapeDtypeStruct((_SEQ, 2, 16, 8, 128), jnp.float32),
    scratch_types=[
        pltpu.VMEM((_WIN * _NH,), jnp.float32),
        pltpu.VMEM((2, 8, _WINP), jnp.float32),
        pltpu.SemaphoreType.DMA,
    ],
    compiler_params=pltpu.CompilerParams(
        use_tc_tiling_on_sc=False, needs_layout_passes=False
    ),
)
def _rel_embed(tab_hbm, out_hbm, win_v, revt_v, ssem):
    wid = lax.axis_index("s") * _NC + lax.axis_index("c")
    m = wid % 8
    q = wid // 8
    t0 = m + 512 * q

    pltpu.sync_copy(tab_hbm.at[pl.ds(t0 * _NH, _WIN * _NH)], win_v)

    lane = lax.iota(jnp.int32, 16)
    off0 = lane * _NH

    @plsc.parallel_loop(0, _WINP // 16, step=1, unroll=2)
    def _build(s):
        w0 = s * 16
        base = jnp.maximum(
            jnp.full((16,), (_WIN - 1 - w0) * _NH, jnp.int32) - off0, 0
        )
        for h in range(16):
            vec = plsc.load_gather(win_v, [base + h])
            revt_v[h // 8, h % 8, pl.ds(pl.multiple_of(w0, 16), 16)] = vec

    def _scatter(t, _):
        o = (_WIN - _SEQ) - 8 * t
        i = t0 + 8 * t
        handles = []
        for ab in range(32):
            a = ab // 16
            b = ab % 16
            handles.append(
                pltpu.async_copy(
                    revt_v.at[a, :, pl.ds(pl.multiple_of(o + 128 * b, 8), 128)],
                    out_hbm.at[i, a, b],
                    ssem,
                )
            )
        for h in handles:
            h.wait()
        return 0

    lax.fori_loop(0, _ROWS_PER_W, _scatter, 0)


def kernel(rel_table, seq_len):
    del seq_len
    padded = jnp.concatenate(
        [rel_table, jnp.zeros((_TBLP - _TBL, _NH), rel_table.dtype)], axis=0
    )
    five = _rel_embed(padded.reshape(-1))
    return five.transpose(0, 2, 4, 1, 3).reshape(_SEQ, _SEQ, _NH)
